# Initial kernel scaffold; baseline (speedup 1.0000x reference)
import jax
import jax.numpy as jnp
from jax.experimental import pallas as pl


def kernel(node_indexes, A, embedding_states, fc_w, fc_b):
    raise NotImplementedError("write your pallas kernel here")


# speedup vs baseline: 290.0278x; 290.0278x over previous
import functools

import jax
import jax.numpy as jnp
from jax.experimental import pallas as pl
from jax.experimental.pallas import tpu as pltpu
from jax.experimental.pallas import tpu_sc as plsc


def _sc_gather(table, idx):
    V, D = table.shape
    B = idx.shape[0]
    info = plsc.get_sparse_core_info()
    NC, NS = info.num_cores, info.num_subcores
    NW = NC * NS
    b_per_w = B // NW
    mesh = plsc.VectorSubcoreMesh(core_axis_name="c", subcore_axis_name="s")

    @functools.partial(
        pl.kernel,
        mesh=mesh,
        out_type=jax.ShapeDtypeStruct((B, D), jnp.float32),
        scratch_types=[
            pltpu.VMEM((b_per_w,), jnp.int32),
            pltpu.VMEM((b_per_w, D), jnp.float32),
            pltpu.SemaphoreType.DMA,
        ],
    )
    def k(idx_hbm, table_hbm, out_hbm, idx_v, rows_v, sem):
        wid = jax.lax.axis_index("s") * NC + jax.lax.axis_index("c")
        base = wid * b_per_w
        pltpu.sync_copy(idx_hbm.at[pl.ds(base, b_per_w)], idx_v)
        pltpu.async_copy(table_hbm.at[idx_v], rows_v, sem).wait()
        pltpu.sync_copy(rows_v, out_hbm.at[pl.ds(base, b_per_w)])

    return k(idx, table)


def _attn_body(idx_ref, a_hbm, e_ref, cen_ref, w1_ref, w2_ref, b_ref,
               out_ref, rowbuf, srow, sems, *, rb):
    g = pl.program_id(0)
    ng = pl.num_programs(0)

    def start_copies(step, slot):
        base = step * rb
        for j in range(rb):
            pltpu.make_async_copy(
                a_hbm.at[idx_ref[base + j]], rowbuf.at[slot, j], sems.at[slot, j]
            ).start()

    def wait_copies(slot):
        for j in range(rb):
            pltpu.make_async_copy(
                a_hbm.at[0], rowbuf.at[slot, j], sems.at[slot, j]
            ).wait()

    @pl.when(g == 0)
    def _():
        start_copies(0, 0)
        srow[...] = jax.lax.dot_general(
            w2_ref[...], e_ref[...],
            (((1,), (1,)), ((), ())),
            preferred_element_type=jnp.float32,
        ) + b_ref[0]

    @pl.when(g + 1 < ng)
    def _():
        start_copies(g + 1, (g + 1) % 2)

    slot = g % 2
    wait_copies(slot)
    rows = rowbuf[slot]
    centers = cen_ref[...]
    c = jnp.sum(centers * w1_ref[...], axis=1, keepdims=True)
    x = c + srow[...]
    att = jnp.maximum(x, 0.2 * x)
    w = jnp.where(rows > 0, att, 0.0)
    out_ref[...] = jnp.dot(
        w, e_ref[...], preferred_element_type=jnp.float32
    ) + centers


def kernel(node_indexes, A, embedding_states, fc_w, fc_b):
    B = node_indexes.shape[0]
    N, D = embedding_states.shape
    idx = node_indexes.astype(jnp.int32)
    w1 = fc_w[:, :D]
    w2 = fc_w[:, D:]
    b = fc_b.astype(jnp.float32)

    centers = _sc_gather(embedding_states, idx)

    rb = 128
    while B % rb:
        rb //= 2
    grid = (B // rb,)

    out = pl.pallas_call(
        functools.partial(_attn_body, rb=rb),
        grid=grid,
        in_specs=[
            pl.BlockSpec(memory_space=pltpu.SMEM),
            pl.BlockSpec(memory_space=pl.ANY),
            pl.BlockSpec((N, D), lambda g: (0, 0)),
            pl.BlockSpec((rb, D), lambda g: (g, 0)),
            pl.BlockSpec((1, D), lambda g: (0, 0)),
            pl.BlockSpec((1, D), lambda g: (0, 0)),
            pl.BlockSpec(memory_space=pltpu.SMEM),
        ],
        out_specs=pl.BlockSpec((rb, D), lambda g: (g, 0)),
        out_shape=jax.ShapeDtypeStruct((B, D), jnp.float32),
        scratch_shapes=[
            pltpu.VMEM((2, rb, N), jnp.float32),
            pltpu.VMEM((1, N), jnp.float32),
            pltpu.SemaphoreType.DMA((2, rb)),
        ],
        compiler_params=pltpu.CompilerParams(
            dimension_semantics=("arbitrary",),
        ),
    )(idx, A, embedding_states, centers, w1, w2, b)
    return out
